# baseline (speedup 1.0000x reference)
import functools

import jax
import jax.numpy as jnp
from jax import lax
from jax.experimental import pallas as pl
from jax.experimental.pallas import tpu as pltpu
from jax.experimental.pallas import tpu_sc as plsc

_NC = 2
_NS = 16
_NW = _NC * _NS

_B = 16384
_D = 64
_CHUNK = 128
_ROWS = _B // _CHUNK
_ROWS_PER_W = _ROWS // _NW
_B_PER_W = _B // _NW


def _gather_body(table_hbm, idx_hbm, out_hbm, idx_v, rows_v, sem):
    wid = lax.axis_index("s") * _NC + lax.axis_index("c")
    row0 = wid * _ROWS_PER_W
    pltpu.sync_copy(idx_hbm.at[pl.ds(row0, _ROWS_PER_W)], idx_v)
    copies = [
        pltpu.async_copy(
            table_hbm.at[idx_v.at[j]],
            rows_v.at[pl.ds(j * _CHUNK, _CHUNK)],
            sem,
        )
        for j in range(_ROWS_PER_W)
    ]
    for c in copies:
        c.wait()
    pltpu.sync_copy(rows_v, out_hbm.at[pl.ds(wid * _B_PER_W, _B_PER_W)])


@jax.jit
def _gather(table, idx_rows):
    mesh = plsc.VectorSubcoreMesh(core_axis_name="c", subcore_axis_name="s")
    return pl.kernel(
        _gather_body,
        mesh=mesh,
        out_type=jax.ShapeDtypeStruct((_B, _D), jnp.float32),
        scratch_types=[
            pltpu.VMEM((_ROWS_PER_W, _CHUNK), jnp.int32),
            pltpu.VMEM((_B_PER_W, _D), jnp.float32),
            pltpu.SemaphoreType.DMA,
        ],
        compiler_params=pltpu.CompilerParams(use_tc_tiling_on_sc=False),
    )(table, idx_rows)


def kernel(tensor, inds):
    idx_rows = jnp.reshape(inds, (_ROWS, _CHUNK))
    return _gather(tensor, idx_rows)


# speedup vs baseline: 1.0342x; 1.0342x over previous
import functools

import jax
import jax.numpy as jnp
from jax import lax
from jax.experimental import pallas as pl
from jax.experimental.pallas import tpu as pltpu
from jax.experimental.pallas import tpu_sc as plsc

_NC = 2
_NS = 16
_NW = _NC * _NS

_B = 16384
_D = 64
_B_PER_W = _B // _NW


def _gather_body(table_hbm, idx_hbm, out_hbm, idx_v, sem):
    wid = lax.axis_index("s") * _NC + lax.axis_index("c")
    base = wid * _B_PER_W
    pltpu.sync_copy(idx_hbm.at[pl.ds(base, _B_PER_W)], idx_v)

    def issue(g, carry):
        v = idx_v[pl.ds(g * 16, 16)]
        for k in range(16):
            pltpu.async_copy(
                table_hbm.at[pl.ds(v[k], 1)],
                out_hbm.at[pl.ds(base + g * 16 + k, 1)],
                sem,
            )
        return carry

    lax.fori_loop(0, _B_PER_W // 16, issue, 0)

    def drain(j, carry):
        pltpu.make_async_copy(
            table_hbm.at[pl.ds(0, 1)],
            out_hbm.at[pl.ds(base + j, 1)],
            sem,
        ).wait()
        return carry

    lax.fori_loop(0, _B_PER_W, drain, 0)


@jax.jit
def _gather(table, idx):
    mesh = plsc.VectorSubcoreMesh(core_axis_name="c", subcore_axis_name="s")
    return pl.kernel(
        _gather_body,
        mesh=mesh,
        out_type=jax.ShapeDtypeStruct((_B, _D), jnp.float32),
        scratch_types=[
            pltpu.VMEM((_B_PER_W,), jnp.int32),
            pltpu.SemaphoreType.DMA,
        ],
        compiler_params=pltpu.CompilerParams(use_tc_tiling_on_sc=True),
    )(table, idx)


def kernel(tensor, inds):
    return _gather(tensor, jnp.squeeze(inds, axis=1))


# speedup vs baseline: 1.7258x; 1.6687x over previous
import functools

import jax
import jax.numpy as jnp
from jax import lax
from jax.experimental import pallas as pl
from jax.experimental.pallas import tpu as pltpu
from jax.experimental.pallas import tpu_sc as plsc

_NC = 2
_NS = 16
_NW = _NC * _NS

_B = 16384
_D = 64
_B_PER_W = _B // _NW


def _gather_body(table_hbm, idx_hbm, out_hbm, idx_v, rows_v, sem):
    wid = lax.axis_index("s") * _NC + lax.axis_index("c")
    base = wid * _B_PER_W
    pltpu.sync_copy(idx_hbm.at[pl.ds(base, _B_PER_W)], idx_v)

    def issue(g, carry):
        v = idx_v[pl.ds(g * 16, 16)]
        for k in range(16):
            pltpu.async_copy(
                table_hbm.at[pl.ds(v[k], 1)],
                rows_v.at[pl.ds(g * 16 + k, 1)],
                sem,
            )
        return carry

    lax.fori_loop(0, _B_PER_W // 16, issue, 0)

    def drain(j, carry):
        pltpu.make_async_copy(
            table_hbm.at[pl.ds(0, 1)],
            rows_v.at[pl.ds(j, 1)],
            sem,
        ).wait()
        return carry

    lax.fori_loop(0, _B_PER_W, drain, 0)

    pltpu.sync_copy(rows_v, out_hbm.at[pl.ds(base, _B_PER_W)])


@jax.jit
def _gather(table, idx):
    mesh = plsc.VectorSubcoreMesh(core_axis_name="c", subcore_axis_name="s")
    return pl.kernel(
        _gather_body,
        mesh=mesh,
        out_type=jax.ShapeDtypeStruct((_B, _D), jnp.float32),
        scratch_types=[
            pltpu.VMEM((_B_PER_W,), jnp.int32),
            pltpu.VMEM((_B_PER_W, _D), jnp.float32),
            pltpu.SemaphoreType.DMA,
        ],
        compiler_params=pltpu.CompilerParams(use_tc_tiling_on_sc=True),
    )(table, idx)


def kernel(tensor, inds):
    return _gather(tensor, jnp.squeeze(inds, axis=1))


# speedup vs baseline: 2.5891x; 1.5003x over previous
import functools

import jax
import jax.numpy as jnp
from jax import lax
from jax.experimental import pallas as pl
from jax.experimental.pallas import tpu as pltpu
from jax.experimental.pallas import tpu_sc as plsc

_NC = 2
_NS = 16
_NW = _NC * _NS

_B = 16384
_D = 64
_N = 1000000
_B_PER_W = _B // _NW
_NGRP = _B_PER_W // 16
_DEPTH = 4


def _gather_body(tt_hbm, idx_hbm, out_hbm, idx_v, q_v, r_v, stg_v,
                 b0, b1, b2, b3, s0, s1, s2, s3):
    bufs = (b0, b1, b2, b3)
    sems = (s0, s1, s2, s3)
    wid = lax.axis_index("s") * _NC + lax.axis_index("c")
    base = wid * _B_PER_W
    pltpu.sync_copy(idx_hbm.at[pl.ds(base, _B_PER_W)], idx_v)

    def prep(jg, carry):
        v = idx_v[pl.ds(jg * 16, 16)]
        q_v[pl.ds(jg * 16, 16)] = lax.shift_right_logical(v, 7)
        r_v[pl.ds(jg * 16, 16)] = lax.bitwise_and(v, 127)
        return carry

    lax.fori_loop(0, _NGRP, prep, 0)

    def fetch(q, slot):
        pltpu.async_copy(
            tt_hbm.at[:, pl.ds(q * 128, 128)],
            bufs[slot],
            sems[slot],
        )

    qhead = q_v[pl.ds(0, 16)]
    for t in range(_DEPTH - 1):
        fetch(qhead[t], t % _DEPTH)

    lanes = lax.iota(jnp.int32, 16)
    z16 = jnp.zeros((16,), jnp.int32)

    def group(jg, carry):
        qv = q_v[pl.ds(jg * 16, 16)]
        rv = r_v[pl.ds(jg * 16, 16)]
        jg_next = lax.min(jg + 1, _NGRP - 1)
        qnext = q_v[pl.ds(jg_next * 16, 16)]
        for k in range(16):
            slot = k % _DEPTH
            if k + _DEPTH - 1 < 16:
                qa = qv[k + _DEPTH - 1]
            else:
                qa = qnext[k + _DEPTH - 1 - 16]
            fetch(qa, (k + _DEPTH - 1) % _DEPTH)
            pltpu.make_async_copy(
                tt_hbm.at[:, pl.ds(0, 128)],
                bufs[slot],
                sems[slot],
            ).wait()
            rb = z16 + rv[k]
            tb = z16 + (jg * 16 + k)
            for m in range(_D // 16):
                c16 = lanes + m * 16
                vals = plsc.load_gather(bufs[slot], [c16, rb])
                plsc.store_scatter(stg_v, [c16, tb], vals)
        return carry

    lax.fori_loop(0, _NGRP, group, 0)

    for t in range(_DEPTH - 1):
        slot = (_B_PER_W + t) % _DEPTH
        pltpu.make_async_copy(
            tt_hbm.at[:, pl.ds(0, 128)],
            bufs[slot],
            sems[slot],
        ).wait()

    pltpu.sync_copy(stg_v, out_hbm.at[:, pl.ds(base, _B_PER_W)])


@jax.jit
def _gather(tt, idx):
    mesh = plsc.VectorSubcoreMesh(core_axis_name="c", subcore_axis_name="s")
    return pl.kernel(
        _gather_body,
        mesh=mesh,
        out_type=jax.ShapeDtypeStruct((_D, _B), jnp.float32),
        scratch_types=[
            pltpu.VMEM((_B_PER_W,), jnp.int32),
            pltpu.VMEM((_B_PER_W,), jnp.int32),
            pltpu.VMEM((_B_PER_W,), jnp.int32),
            pltpu.VMEM((_D, _B_PER_W), jnp.float32),
            pltpu.VMEM((_D, 128), jnp.float32),
            pltpu.VMEM((_D, 128), jnp.float32),
            pltpu.VMEM((_D, 128), jnp.float32),
            pltpu.VMEM((_D, 128), jnp.float32),
            pltpu.SemaphoreType.DMA,
            pltpu.SemaphoreType.DMA,
            pltpu.SemaphoreType.DMA,
            pltpu.SemaphoreType.DMA,
        ],
        compiler_params=pltpu.CompilerParams(
            use_tc_tiling_on_sc=True, needs_layout_passes=False),
    )(tt, idx)


def kernel(tensor, inds):
    out_t = _gather(tensor.T, jnp.squeeze(inds, axis=1))
    return out_t.T


# speedup vs baseline: 3.0043x; 1.1603x over previous
import functools

import jax
import jax.numpy as jnp
from jax import lax
from jax.experimental import pallas as pl
from jax.experimental.pallas import tpu as pltpu
from jax.experimental.pallas import tpu_sc as plsc

_NC = 2
_NS = 16
_NW = _NC * _NS

_B = 16384
_D = 64
_N = 1000000
_B_PER_W = _B // _NW
_NGRP = _B_PER_W // 16
_DEPTH = 8


def _gather_body(tt_hbm, idx_hbm, out_hbm, idx_v, q_v, r_v, stg_v,
                 b0, b1, b2, b3, b4, b5, b6, b7,
                 s0, s1, s2, s3, s4, s5, s6, s7):
    bufs = (b0, b1, b2, b3, b4, b5, b6, b7)
    sems = (s0, s1, s2, s3, s4, s5, s6, s7)
    wid = lax.axis_index("s") * _NC + lax.axis_index("c")
    base = wid * _B_PER_W
    pltpu.sync_copy(idx_hbm.at[pl.ds(base, _B_PER_W)], idx_v)

    def prep(jg, carry):
        v = idx_v[pl.ds(jg * 16, 16)]
        q_v[pl.ds(jg * 16, 16)] = lax.shift_right_logical(v, 7)
        r_v[pl.ds(jg * 16, 16)] = lax.bitwise_and(v, 127)
        return carry

    lax.fori_loop(0, _NGRP, prep, 0)

    def fetch(q, slot):
        pltpu.async_copy(
            tt_hbm.at[:, pl.ds(q * 128, 128)],
            bufs[slot],
            sems[slot],
        )

    qhead = q_v[pl.ds(0, 16)]
    for t in range(_DEPTH - 1):
        fetch(qhead[t], t % _DEPTH)

    lanes = lax.iota(jnp.int32, 16)
    z16 = jnp.zeros((16,), jnp.int32)

    def group(jg, carry):
        qv = q_v[pl.ds(jg * 16, 16)]
        rv = r_v[pl.ds(jg * 16, 16)]
        jg_next = lax.min(jg + 1, _NGRP - 1)
        qnext = q_v[pl.ds(jg_next * 16, 16)]
        for k in range(16):
            slot = k % _DEPTH
            if k + _DEPTH - 1 < 16:
                qa = qv[k + _DEPTH - 1]
            else:
                qa = qnext[k + _DEPTH - 1 - 16]
            fetch(qa, (k + _DEPTH - 1) % _DEPTH)
            pltpu.make_async_copy(
                tt_hbm.at[:, pl.ds(0, 128)],
                bufs[slot],
                sems[slot],
            ).wait()
            rb = z16 + rv[k]
            tb = z16 + (jg * 16 + k)
            for m in range(_D // 16):
                c16 = lanes + m * 16
                vals = plsc.load_gather(bufs[slot], [c16, rb])
                plsc.store_scatter(stg_v, [c16, tb], vals)
        return carry

    lax.fori_loop(0, _NGRP, group, 0)

    for t in range(_DEPTH - 1):
        slot = (_B_PER_W + t) % _DEPTH
        pltpu.make_async_copy(
            tt_hbm.at[:, pl.ds(0, 128)],
            bufs[slot],
            sems[slot],
        ).wait()

    pltpu.sync_copy(stg_v, out_hbm.at[:, pl.ds(base, _B_PER_W)])


@jax.jit
def _gather(tt, idx):
    mesh = plsc.VectorSubcoreMesh(core_axis_name="c", subcore_axis_name="s")
    return pl.kernel(
        _gather_body,
        mesh=mesh,
        out_type=jax.ShapeDtypeStruct((_D, _B), jnp.float32),
        scratch_types=[
            pltpu.VMEM((_B_PER_W,), jnp.int32),
            pltpu.VMEM((_B_PER_W,), jnp.int32),
            pltpu.VMEM((_B_PER_W,), jnp.int32),
            pltpu.VMEM((_D, _B_PER_W), jnp.float32),
            pltpu.VMEM((_D, 128), jnp.float32),
            pltpu.VMEM((_D, 128), jnp.float32),
            pltpu.VMEM((_D, 128), jnp.float32),
            pltpu.VMEM((_D, 128), jnp.float32),
            pltpu.VMEM((_D, 128), jnp.float32),
            pltpu.VMEM((_D, 128), jnp.float32),
            pltpu.VMEM((_D, 128), jnp.float32),
            pltpu.VMEM((_D, 128), jnp.float32),
            pltpu.SemaphoreType.DMA,
            pltpu.SemaphoreType.DMA,
            pltpu.SemaphoreType.DMA,
            pltpu.SemaphoreType.DMA,
            pltpu.SemaphoreType.DMA,
            pltpu.SemaphoreType.DMA,
            pltpu.SemaphoreType.DMA,
            pltpu.SemaphoreType.DMA,
        ],
        compiler_params=pltpu.CompilerParams(
            use_tc_tiling_on_sc=True, needs_layout_passes=False),
    )(tt, idx)


def kernel(tensor, inds):
    out_t = _gather(tensor.T, jnp.squeeze(inds, axis=1))
    return out_t.T


# speedup vs baseline: 3.0104x; 1.0020x over previous
import functools

import jax
import jax.numpy as jnp
from jax import lax
from jax.experimental import pallas as pl
from jax.experimental.pallas import tpu as pltpu
from jax.experimental.pallas import tpu_sc as plsc

_NC = 2
_NS = 16
_NW = _NC * _NS

_B = 16384
_D = 64
_N = 1000000
_B_PER_W = _B // _NW
_NGRP = _B_PER_W // 16
_DEPTH = 8


def _gather_body(tt_hbm, idx_hbm, out_hbm, idx_v, q_v, r_v, stg_v,
                 b0, b1, b2, b3, b4, b5, b6, b7,
                 s0, s1, s2, s3, s4, s5, s6, s7):
    bufs = (b0, b1, b2, b3, b4, b5, b6, b7)
    sems = (s0, s1, s2, s3, s4, s5, s6, s7)
    wid = lax.axis_index("s") * _NC + lax.axis_index("c")
    base = wid * _B_PER_W
    pltpu.sync_copy(idx_hbm.at[pl.ds(base, _B_PER_W)], idx_v)

    def prep(jg, carry):
        v = idx_v[pl.ds(jg * 16, 16)]
        q_v[pl.ds(jg * 16, 16)] = lax.shift_right_logical(v, 7)
        r_v[pl.ds(jg * 16, 16)] = lax.bitwise_and(v, 127)
        return carry

    lax.fori_loop(0, _NGRP, prep, 0)

    def fetch(q, slot):
        pltpu.async_copy(
            tt_hbm.at[pl.ds(0, 32), pl.ds(q * 128, 128)],
            bufs[slot].at[pl.ds(0, 32)],
            sems[slot],
        )
        pltpu.async_copy(
            tt_hbm.at[pl.ds(32, 32), pl.ds(q * 128, 128)],
            bufs[slot].at[pl.ds(32, 32)],
            sems[slot],
        )

    def wait(slot):
        for h in range(2):
            pltpu.make_async_copy(
                tt_hbm.at[pl.ds(h * 32, 32), pl.ds(0, 128)],
                bufs[slot].at[pl.ds(h * 32, 32)],
                sems[slot],
            ).wait()

    qhead = q_v[pl.ds(0, 16)]
    for t in range(_DEPTH - 1):
        fetch(qhead[t], t % _DEPTH)

    lanes = lax.iota(jnp.int32, 16)
    z16 = jnp.zeros((16,), jnp.int32)

    def group(jg, carry):
        qv = q_v[pl.ds(jg * 16, 16)]
        rv = r_v[pl.ds(jg * 16, 16)]
        jg_next = lax.min(jg + 1, _NGRP - 1)
        qnext = q_v[pl.ds(jg_next * 16, 16)]
        for k in range(16):
            slot = k % _DEPTH
            if k + _DEPTH - 1 < 16:
                qa = qv[k + _DEPTH - 1]
            else:
                qa = qnext[k + _DEPTH - 1 - 16]
            fetch(qa, (k + _DEPTH - 1) % _DEPTH)
            wait(slot)
            rb = z16 + rv[k]
            tb = z16 + (jg * 16 + k)
            for m in range(_D // 16):
                c16 = lanes + m * 16
                vals = plsc.load_gather(bufs[slot], [c16, rb])
                plsc.store_scatter(stg_v, [c16, tb], vals)
        return carry

    lax.fori_loop(0, _NGRP, group, 0)

    for t in range(_DEPTH - 1):
        wait((_B_PER_W + t) % _DEPTH)

    pltpu.sync_copy(stg_v, out_hbm.at[:, pl.ds(base, _B_PER_W)])


@jax.jit
def _gather(tt, idx):
    mesh = plsc.VectorSubcoreMesh(core_axis_name="c", subcore_axis_name="s")
    return pl.kernel(
        _gather_body,
        mesh=mesh,
        out_type=jax.ShapeDtypeStruct((_D, _B), jnp.float32),
        scratch_types=[
            pltpu.VMEM((_B_PER_W,), jnp.int32),
            pltpu.VMEM((_B_PER_W,), jnp.int32),
            pltpu.VMEM((_B_PER_W,), jnp.int32),
            pltpu.VMEM((_D, _B_PER_W), jnp.float32),
            pltpu.VMEM((_D, 128), jnp.float32),
            pltpu.VMEM((_D, 128), jnp.float32),
            pltpu.VMEM((_D, 128), jnp.float32),
            pltpu.VMEM((_D, 128), jnp.float32),
            pltpu.VMEM((_D, 128), jnp.float32),
            pltpu.VMEM((_D, 128), jnp.float32),
            pltpu.VMEM((_D, 128), jnp.float32),
            pltpu.VMEM((_D, 128), jnp.float32),
            pltpu.SemaphoreType.DMA,
            pltpu.SemaphoreType.DMA,
            pltpu.SemaphoreType.DMA,
            pltpu.SemaphoreType.DMA,
            pltpu.SemaphoreType.DMA,
            pltpu.SemaphoreType.DMA,
            pltpu.SemaphoreType.DMA,
            pltpu.SemaphoreType.DMA,
        ],
        compiler_params=pltpu.CompilerParams(
            use_tc_tiling_on_sc=True, needs_layout_passes=False),
    )(tt, idx)


def kernel(tensor, inds):
    out_t = _gather(tensor.T, jnp.squeeze(inds, axis=1))
    return out_t.T
